# baseline (speedup 1.0000x reference)
import functools

import jax
import jax.numpy as jnp
from jax import lax
from jax.experimental import pallas as pl
from jax.experimental.pallas import tpu as pltpu
from jax.experimental.pallas import tpu_sc as plsc

_F = 26
_V = 100000
_D = 32
_B = 16384

_NC = 2
_NS = 16
_NW = _NC * _NS
_HB = _B // 2
_LANES = 16


def _sc_gather(planes, indices):
    mesh = plsc.VectorSubcoreMesh(core_axis_name="c", subcore_axis_name="s")

    @functools.partial(
        pl.kernel,
        mesh=mesh,
        out_type=jax.ShapeDtypeStruct((_F * _D, _B), jnp.float32),
        compiler_params=pltpu.CompilerParams(needs_layout_passes=False),
        scratch_types=[
            pltpu.VMEM((_V,), jnp.float32),
            pltpu.VMEM((_HB,), jnp.int32),
            pltpu.VMEM((2, _HB), jnp.float32),
            pltpu.SemaphoreType.DMA,
        ],
    )
    def k(tbl_hbm, idx_hbm, out_hbm, plane_v, idx_v, out_v, sem):
        s = lax.axis_index("s")
        c = lax.axis_index("c")
        d = c * _NS + s
        shift = (d // 8) * 7 % _F

        pending = [None, None]
        for j in range(_F):
            f = (j + shift) % _F
            row = f * _D + d
            pltpu.sync_copy(tbl_hbm.at[row], plane_v)
            for h in range(_B // _HB):
                pltpu.sync_copy(idx_hbm.at[f, pl.ds(h * _HB, _HB)], idx_v)
                if pending[h] is not None:
                    pending[h].wait()

                @plsc.parallel_loop(0, _HB, step=_LANES, unroll=8)
                def gloop(i, h=h):
                    iv = idx_v[pl.ds(i, _LANES)]
                    out_v[h, pl.ds(i, _LANES)] = plsc.load_gather(
                        plane_v, [iv]
                    )

                pending[h] = pltpu.async_copy(
                    out_v.at[h], out_hbm.at[row, pl.ds(h * _HB, _HB)], sem
                )
        for cp in pending:
            cp.wait()

    return k(planes, indices)


def kernel(indices, tables):
    planes = jnp.transpose(tables, (0, 2, 1)).reshape(_F * _D, _V)
    out_fd_b = _sc_gather(planes, indices)
    return jnp.transpose(out_fd_b.reshape(_F, _D, _B), (2, 0, 1))


# speedup vs baseline: 1.1019x; 1.1019x over previous
import functools

import jax
import jax.numpy as jnp
from jax import lax
from jax.experimental import pallas as pl
from jax.experimental.pallas import tpu as pltpu
from jax.experimental.pallas import tpu_sc as plsc

_F = 26
_V = 100000
_D = 32
_B = 16384

_NC = 2
_NS = 16
_NW = _NC * _NS
_HB = _B // 4
_LANES = 16
_PHASES = [(0, list(range(0, 8))), (8, list(range(8, 16))),
           (16, list(range(16, 24))), (24, list(range(24, 26)))]


def _sc_gather(planes, indices):
    mesh = plsc.VectorSubcoreMesh(core_axis_name="c", subcore_axis_name="s")

    @functools.partial(
        pl.kernel,
        mesh=mesh,
        out_type=jax.ShapeDtypeStruct((_F * _D, _B), jnp.float32),
        compiler_params=pltpu.CompilerParams(needs_layout_passes=False),
        scratch_types=[
            pltpu.VMEM((_V,), jnp.float32),
            pltpu.VMEM((_HB,), jnp.int32),
            pltpu.VMEM((2, _HB), jnp.float32),
            pltpu.VMEM_SHARED((2, 8, _B), jnp.int32),
            pltpu.SemaphoreType.DMA,
            pltpu.SemaphoreType.DMA,
        ],
    )
    def k(tbl_hbm, idx_hbm, out_hbm, plane_v, idx_v, out_v, idx_sh, sem,
          stage_sem):
        s = lax.axis_index("s")
        c = lax.axis_index("c")
        d = c * _NS + s
        bandshift = d // 8
        pending = [None, None]

        @pl.when(s == 0)
        def _():
            pltpu.sync_copy(idx_hbm.at[pl.ds(0, 8)], idx_sh.at[0])

        for p, (start, fields) in enumerate(_PHASES):
            pb = p % 2

            @pl.when(s == 0)
            def _(p=p, pb=pb):
                if p > 0:
                    st, _fs = _PHASES[p]
                    pltpu.make_async_copy(
                        idx_hbm.at[pl.ds(st, 8)], idx_sh.at[pb], stage_sem
                    ).wait()

            plsc.subcore_barrier()

            @pl.when(s == 0)
            def _(p=p, pb=pb):
                if p + 1 < len(_PHASES):
                    st, _fs = _PHASES[p + 1]
                    pltpu.async_copy(
                        idx_hbm.at[pl.ds(st, 8)], idx_sh.at[1 - pb],
                        stage_sem,
                    )

            nf = len(fields)
            base = fields[0]
            for jj in range(nf):
                f = base + (jj + bandshift) % nf
                row = f * _D + d
                pltpu.sync_copy(tbl_hbm.at[row], plane_v)
                for h in range(_B // _HB):
                    hb = h % 2
                    pltpu.sync_copy(
                        idx_sh.at[pb, f - start, pl.ds(h * _HB, _HB)], idx_v
                    )
                    if pending[hb] is not None:
                        pending[hb].wait()

                    @plsc.parallel_loop(0, _HB, step=_LANES, unroll=8)
                    def gloop(i, hb=hb):
                        iv = idx_v[pl.ds(i, _LANES)]
                        out_v[hb, pl.ds(i, _LANES)] = plsc.load_gather(
                            plane_v, [iv]
                        )

                    pending[hb] = pltpu.async_copy(
                        out_v.at[hb], out_hbm.at[row, pl.ds(h * _HB, _HB)],
                        sem,
                    )
        for cp in pending:
            cp.wait()

    return k(planes, indices)


def kernel(indices, tables):
    planes = jnp.transpose(tables, (0, 2, 1)).reshape(_F * _D, _V)
    idx_pad = jnp.pad(indices, ((0, 32 - _F), (0, 0)))
    out_fd_b = _sc_gather(planes, idx_pad)
    return jnp.transpose(out_fd_b.reshape(_F, _D, _B), (2, 0, 1))
